# baseline (speedup 1.0000x reference)
import functools

import jax
import jax.numpy as jnp
from jax import lax
from jax.experimental import pallas as pl
from jax.experimental.pallas import tpu as pltpu
from jax.experimental.pallas import tpu_sc as plsc

_NUM_ROWS = 100000
_ROWS_PAD = 100096
_DIM = 64
_BATCH = 16384

_info = plsc.get_sparse_core_info()
_NC, _NS = _info.num_cores, _info.num_subcores
_NW = _NC * _NS
_D_PER_W = _DIM // _NW
_CHUNK = 8192
_NCHUNK = _BATCH // _CHUNK

_mesh = plsc.VectorSubcoreMesh(core_axis_name="c", subcore_axis_name="s")


@functools.partial(
    pl.kernel,
    mesh=_mesh,
    out_type=jax.ShapeDtypeStruct((_DIM, _BATCH), jnp.float32),
    scratch_types=[
        pltpu.VMEM((_ROWS_PAD,), jnp.float32),
        pltpu.VMEM((_BATCH,), jnp.int32),
        pltpu.VMEM((_CHUNK,), jnp.float32),
        pltpu.SemaphoreType.DMA,
    ],
    compiler_params=pltpu.CompilerParams(
        needs_layout_passes=False, disable_bounds_checks=True
    ),
)
def _gather_kernel(table_hbm, idx_hbm, out_hbm, seg_v, idx_v, val_v, sem):
    wid = lax.axis_index("s") * _NC + lax.axis_index("c")
    pltpu.sync_copy(idx_hbm, idx_v)

    _UNROLL = 16

    for t in range(_D_PER_W):
        d = wid * _D_PER_W + t
        pltpu.sync_copy(table_hbm.at[d], seg_v)
        for c in range(_NCHUNK):

            def body(i, carry):
                base = i * (16 * _UNROLL)
                for u in range(_UNROLL):
                    lanes = idx_v[pl.ds(c * _CHUNK + base + u * 16, 16)]
                    vals = plsc.load_gather(seg_v, [lanes])
                    val_v[pl.ds(base + u * 16, 16)] = vals
                return carry

            lax.fori_loop(0, _CHUNK // (16 * _UNROLL), body, 0)
            pltpu.sync_copy(val_v, out_hbm.at[d, pl.ds(c * _CHUNK, _CHUNK)])


def kernel(style_id, embeddings):
    idx = style_id.astype(jnp.int32)
    table_t = jnp.pad(embeddings.T, ((0, 0), (0, _ROWS_PAD - _NUM_ROWS)))
    out_t = _gather_kernel(table_t, idx)
    return out_t.T


# speedup vs baseline: 1.0132x; 1.0132x over previous
import functools

import jax
import jax.numpy as jnp
from jax import lax
from jax.experimental import pallas as pl
from jax.experimental.pallas import tpu as pltpu
from jax.experimental.pallas import tpu_sc as plsc

_NUM_ROWS = 100000
_ROWS_PAD = 100096
_DIM = 64
_BATCH = 16384

_info = plsc.get_sparse_core_info()
_NC, _NS = _info.num_cores, _info.num_subcores
_NW = _NC * _NS
_D_PER_W = _DIM // _NW
_CHUNK = 4096
_NCHUNK = _BATCH // _CHUNK

_mesh = plsc.VectorSubcoreMesh(core_axis_name="c", subcore_axis_name="s")


@functools.partial(
    pl.kernel,
    mesh=_mesh,
    out_type=jax.ShapeDtypeStruct((_DIM, _BATCH), jnp.float32),
    scratch_types=[
        pltpu.VMEM((_ROWS_PAD,), jnp.float32),
        pltpu.VMEM((_BATCH,), jnp.int32),
        pltpu.VMEM((_CHUNK,), jnp.float32),
        pltpu.VMEM((_CHUNK,), jnp.float32),
        pltpu.SemaphoreType.DMA,
        pltpu.SemaphoreType.DMA,
    ],
    compiler_params=pltpu.CompilerParams(
        needs_layout_passes=False, disable_bounds_checks=True
    ),
)
def _gather_kernel(table_hbm, idx_hbm, out_hbm, seg_v, idx_v, val_a, val_b, sem, wsem):
    wid = lax.axis_index("s") * _NC + lax.axis_index("c")
    pltpu.sync_copy(idx_hbm, idx_v)

    _UNROLL = 8

    for t in range(_D_PER_W):
        d = wid * _D_PER_W + t
        pltpu.sync_copy(table_hbm.at[d], seg_v)
        writes = []
        for c in range(_NCHUNK):
            buf = val_a if c % 2 == 0 else val_b
            if len(writes) >= 2:
                writes[-2].wait()

            def body(i, carry):
                base = i * (16 * _UNROLL)
                for u in range(_UNROLL):
                    lanes = idx_v[pl.ds(c * _CHUNK + base + u * 16, 16)]
                    vals = plsc.load_gather(seg_v, [lanes])
                    buf[pl.ds(base + u * 16, 16)] = vals
                return carry

            lax.fori_loop(0, _CHUNK // (16 * _UNROLL), body, 0)
            writes.append(
                pltpu.async_copy(buf, out_hbm.at[d, pl.ds(c * _CHUNK, _CHUNK)], wsem)
            )
        writes[-2].wait()
        writes[-1].wait()


def kernel(style_id, embeddings):
    idx = style_id.astype(jnp.int32)
    table_t = jnp.pad(embeddings.T, ((0, 0), (0, _ROWS_PAD - _NUM_ROWS)))
    out_t = _gather_kernel(table_t, idx)
    return out_t.T
